# baseline (speedup 1.0000x reference)
import functools

import jax
import jax.numpy as jnp
from jax import lax
from jax.experimental import pallas as pl
from jax.experimental.pallas import tpu as pltpu
from jax.experimental.pallas import tpu_sc as plsc

B, C, R, K = 4, 192, 512, 15
NCH = 8
NSEL = 8
LANES = 16
TASKS = B * C
XWORDS = R * K
OWORDS = R * NSEL
GROUPS = OWORDS // LANES
UNROLL = 8


def _make_sc_call():
    info = plsc.get_sparse_core_info()
    nc, ns = info.num_cores, info.num_subcores
    nw = nc * ns
    assert TASKS % nw == 0
    tpw = TASKS // nw

    mesh = plsc.VectorSubcoreMesh(core_axis_name="c", subcore_axis_name="s")

    @functools.partial(
        pl.kernel,
        mesh=mesh,
        compiler_params=pltpu.CompilerParams(needs_layout_passes=False),
        out_type=jax.ShapeDtypeStruct((B * NCH * C, OWORDS), jnp.float32),
        scratch_types=[
            pltpu.VMEM((NCH * LANES,), jnp.float32),
            pltpu.VMEM((LANES,), jnp.int32),
            pltpu.VMEM((LANES,), jnp.float32),
            pltpu.VMEM((XWORDS,), jnp.float32),
            pltpu.VMEM((NCH * OWORDS,), jnp.float32),
            pltpu.VMEM((NCH * LANES,), jnp.int32),
            pltpu.VMEM((NCH * LANES,), jnp.float32),
        ],
    )
    def sc_call(x_hbm, w_hbm, out_hbm, w_v, idxtab, wtab, xin, obuf,
                basetab, wseltab):
        wid = lax.axis_index("s") * nc + lax.axis_index("c")

        pltpu.sync_copy(w_hbm, w_v)

        iota = lax.iota(jnp.int32, LANES)
        n_vec = lax.bitwise_and(iota, NSEL - 1)
        lane_r = lax.shift_right_logical(iota, 3)

        for i in range(NCH):
            row = w_v[pl.ds(i * LANES, LANES)]
            rank = jnp.zeros((LANES,), jnp.int32)
            for j in range(K):
                wj = jnp.full((LANES,), row[j])
                cond = (wj < row) | ((wj == row) & (j < iota))
                rank = rank + cond.astype(jnp.int32)
            sel = rank < NSEL
            sig = 1.0 / (1.0 + jnp.exp(-row))
            plsc.store_scatter(idxtab, [rank], iota, mask=sel)
            plsc.store_scatter(wtab, [rank], sig, mask=sel)
            idx_sel = plsc.load_gather(idxtab, [n_vec])
            w_sel = plsc.load_gather(wtab, [n_vec])
            basetab[pl.ds(i * LANES, LANES)] = lane_r * K + idx_sel
            wseltab[pl.ds(i * LANES, LANES)] = w_sel

        def task_body(t, carry):
            task = wid * tpw + t
            b = task // C
            c = task - b * C
            pltpu.sync_copy(x_hbm.at[task], xin)
            for i in range(NCH):
                obase = i * OWORDS

                def group_body(it, _, i=i, obase=obase):
                    base = basetab[pl.ds(i * LANES, LANES)]
                    wsel = wseltab[pl.ds(i * LANES, LANES)]
                    g0 = it * UNROLL
                    for u in range(UNROLL):
                        g = g0 + u
                        src = base + (2 * K) * g
                        val = plsc.load_gather(xin, [src])
                        obuf[pl.ds(obase + g * LANES, LANES)] = val * wsel
                    return _

                lax.fori_loop(0, GROUPS // UNROLL, group_body, 0)
            orow0 = b * (NCH * C) + c
            for i in range(NCH):
                pltpu.sync_copy(obuf.at[pl.ds(i * OWORDS, OWORDS)],
                                out_hbm.at[orow0 + i * C])
            return carry

        lax.fori_loop(0, tpw, task_body, 0)

    return sc_call


_sc_call = _make_sc_call()


def kernel(X, partition_weights):
    wpad = jnp.concatenate(
        [partition_weights,
         jnp.full((NCH, LANES - K), jnp.inf, jnp.float32)], axis=1)
    x2 = X.reshape(B * C, XWORDS)
    out = _sc_call(x2, wpad.reshape(NCH * LANES))
    return out.reshape(B, NCH * C, R, NSEL)


# speedup vs baseline: 1.6025x; 1.6025x over previous
import functools

import jax
import jax.numpy as jnp
from jax import lax
from jax.experimental import pallas as pl
from jax.experimental.pallas import tpu as pltpu
from jax.experimental.pallas import tpu_sc as plsc

B, C, R, K = 4, 192, 512, 15
NCH = 8
NSEL = 8
LANES = 16
TASKS = B * C
XWORDS = R * K
OWORDS = R * NSEL
GROUPS = OWORDS // LANES
UNROLL = 8


def _make_sc_call():
    info = plsc.get_sparse_core_info()
    nc, ns = info.num_cores, info.num_subcores
    nw = nc * ns
    assert TASKS % nw == 0
    tpw = TASKS // nw
    assert tpw % 2 == 0
    npairs = tpw // 2

    mesh = plsc.VectorSubcoreMesh(core_axis_name="c", subcore_axis_name="s")

    @functools.partial(
        pl.kernel,
        mesh=mesh,
        compiler_params=pltpu.CompilerParams(needs_layout_passes=False),
        out_type=jax.ShapeDtypeStruct((B * NCH * C, OWORDS), jnp.float32),
        scratch_types=[
            pltpu.VMEM((NCH * LANES,), jnp.float32),
            pltpu.VMEM((LANES,), jnp.int32),
            pltpu.VMEM((LANES,), jnp.float32),
            pltpu.VMEM((NCH * LANES,), jnp.int32),
            pltpu.VMEM((NCH * LANES,), jnp.float32),
            pltpu.VMEM((XWORDS,), jnp.float32),
            pltpu.VMEM((XWORDS,), jnp.float32),
            pltpu.VMEM((NCH, OWORDS), jnp.float32),
            pltpu.VMEM((NCH, OWORDS), jnp.float32),
            pltpu.SemaphoreType.DMA,
            pltpu.SemaphoreType.DMA,
            pltpu.SemaphoreType.DMA,
            pltpu.SemaphoreType.DMA,
        ],
    )
    def sc_call(x_hbm, w_hbm, out_hbm, w_v, idxtab, wtab, basetab, wseltab,
                xin0, xin1, ob0, ob1, si0, si1, so0, so1):
        wid = lax.axis_index("s") * nc + lax.axis_index("c")
        t0base = wid * tpw

        pltpu.sync_copy(w_hbm, w_v)

        iota = lax.iota(jnp.int32, LANES)
        n_vec = lax.bitwise_and(iota, NSEL - 1)
        lane_r = lax.shift_right_logical(iota, 3)

        for i in range(NCH):
            row = w_v[pl.ds(i * LANES, LANES)]
            rank = jnp.zeros((LANES,), jnp.int32)
            for j in range(K):
                wj = jnp.full((LANES,), row[j])
                cond = (wj < row) | ((wj == row) & (j < iota))
                rank = rank + cond.astype(jnp.int32)
            sel = rank < NSEL
            sig = 1.0 / (1.0 + jnp.exp(-row))
            plsc.store_scatter(idxtab, [rank], iota, mask=sel)
            plsc.store_scatter(wtab, [rank], sig, mask=sel)
            idx_sel = plsc.load_gather(idxtab, [n_vec])
            w_sel = plsc.load_gather(wtab, [n_vec])
            basetab[pl.ds(i * LANES, LANES)] = lane_r * K + idx_sel
            wseltab[pl.ds(i * LANES, LANES)] = w_sel

        def orow_of(task):
            b = task // C
            c = task - b * C
            return b * (NCH * C) + c

        def compute(xin, ob):
            for i in range(NCH):
                base = basetab[pl.ds(i * LANES, LANES)]
                wsel = wseltab[pl.ds(i * LANES, LANES)]

                @plsc.parallel_loop(0, GROUPS, unroll=UNROLL)
                def group(g, i=i, base=base, wsel=wsel, xin=xin, ob=ob):
                    src = base + (2 * K) * g
                    val = plsc.load_gather(xin, [src])
                    ob[i, pl.ds(g * LANES, LANES)] = val * wsel

        def fire_out(ob, task, so):
            r0 = orow_of(task)
            for i in range(NCH):
                pltpu.async_copy(ob.at[i], out_hbm.at[r0 + i * C], so)

        def drain_out(ob, task, so):
            r0 = orow_of(task)
            for i in range(NCH):
                pltpu.make_async_copy(ob.at[i], out_hbm.at[r0 + i * C],
                                      so).wait()

        pltpu.async_copy(x_hbm.at[t0base], xin0, si0)

        def pair_body(it, carry):
            task0 = t0base + 2 * it
            task1 = task0 + 1
            pltpu.async_copy(x_hbm.at[task1], xin1, si1)
            pltpu.make_async_copy(x_hbm.at[task0], xin0, si0).wait()

            @pl.when(it > 0)
            def _():
                drain_out(ob0, task0 - 2, so0)

            compute(xin0, ob0)
            fire_out(ob0, task0, so0)

            @pl.when(it + 1 < npairs)
            def _():
                pltpu.async_copy(x_hbm.at[task0 + 2], xin0, si0)

            pltpu.make_async_copy(x_hbm.at[task1], xin1, si1).wait()

            @pl.when(it > 0)
            def _():
                drain_out(ob1, task1 - 2, so1)

            compute(xin1, ob1)
            fire_out(ob1, task1, so1)
            return carry

        lax.fori_loop(0, npairs, pair_body, 0)
        drain_out(ob0, t0base + tpw - 2, so0)
        drain_out(ob1, t0base + tpw - 1, so1)

    return sc_call


_sc_call = _make_sc_call()


def kernel(X, partition_weights):
    wpad = jnp.concatenate(
        [partition_weights,
         jnp.full((NCH, LANES - K), jnp.inf, jnp.float32)], axis=1)
    x2 = X.reshape(B * C, XWORDS)
    out = _sc_call(x2, wpad.reshape(NCH * LANES))
    return out.reshape(B, NCH * C, R, NSEL)
